# baseline (speedup 1.0000x reference)
import jax
import jax.numpy as jnp
from jax.experimental import pallas as pl
from jax.experimental.pallas import tpu as pltpu

BM = 512
BK = 512
BR = 512


def _prep_kernel(d_ref, x_ref, xs_ref, rs_ref):
    deg = jnp.max(d_ref[...], axis=1, keepdims=True)
    rs = jax.lax.rsqrt(deg)
    xs_ref[...] = x_ref[...] * rs
    rs_ref[...] = rs


def _layer1_kernel(a_ref, xs_ref, rsk_ref, rsi_ref, w1t_ref, b1_ref,
                   hs_ref, rsdj0_ref, acc, dj0_acc, found):
    k = pl.program_id(1)
    nk = pl.num_programs(1)

    @pl.when(k == 0)
    def _init():
        acc[...] = jnp.zeros_like(acc)
        dj0_acc[...] = jnp.zeros_like(dj0_acc)
        found[...] = jnp.zeros_like(found)

    mask = a_ref[...] != 0
    acc[...] += jnp.dot(mask.astype(jnp.float32), xs_ref[...],
                        preferred_element_type=jnp.float32)

    bm, bk = mask.shape
    iota = jax.lax.broadcasted_iota(jnp.int32, (bm, bk), 1)
    pos = jnp.where(mask, iota, bk)
    local_first = jnp.min(pos, axis=1, keepdims=True)
    has = local_first < bk
    is_new = jnp.logical_and(has, found[...] == 0)
    onehot = jnp.logical_and(iota == local_first, is_new).astype(jnp.float32)
    dj0_acc[...] += jnp.dot(onehot, rsk_ref[...],
                            preferred_element_type=jnp.float32)
    found[...] = jnp.maximum(found[...], has.astype(jnp.int32))

    @pl.when(k == nk - 1)
    def _epilogue():
        agg = acc[...] * dj0_acc[...]
        h = jnp.dot(agg, w1t_ref[...], preferred_element_type=jnp.float32)
        h = h + b1_ref[...]
        h = jnp.where(h > 0, h, 0.01 * h)
        hs_ref[...] = h * rsi_ref[...]
        rsdj0_ref[...] = dj0_acc[...]


def _layer2_kernel(a_ref, hs_ref, rsdj0_ref, w2t_ref, b2_ref, w3t_ref, b3_ref,
                   out_ref, acc):
    k = pl.program_id(1)
    nk = pl.num_programs(1)

    @pl.when(k == 0)
    def _init():
        acc[...] = jnp.zeros_like(acc)

    mask = (a_ref[...] != 0).astype(jnp.float32)
    acc[...] += jnp.dot(mask, hs_ref[...], preferred_element_type=jnp.float32)

    @pl.when(k == nk - 1)
    def _epilogue():
        agg = acc[...] * rsdj0_ref[...]
        h = jnp.dot(agg, w2t_ref[...], preferred_element_type=jnp.float32)
        h = h + b2_ref[...]
        h = jnp.where(h > 0, h, 0.01 * h)
        o = jnp.dot(h, w3t_ref[...], preferred_element_type=jnp.float32)
        o = o + b3_ref[...]
        m = jnp.max(o, axis=1, keepdims=True)
        e = jnp.exp(o - m)
        out_ref[...] = (o - m) - jnp.log(jnp.sum(e, axis=1, keepdims=True))


def kernel(D, X, A, W1, b1, W2, b2, W3, b3):
    n, f = X.shape
    h1 = W1.shape[0]
    h2 = W2.shape[0]
    c = W3.shape[0]
    f32 = jnp.float32

    xs, rsdeg = pl.pallas_call(
        _prep_kernel,
        grid=(n // BR,),
        in_specs=[
            pl.BlockSpec((BR, n), lambda i: (i, 0)),
            pl.BlockSpec((BR, f), lambda i: (i, 0)),
        ],
        out_specs=[
            pl.BlockSpec((BR, f), lambda i: (i, 0)),
            pl.BlockSpec((BR, 1), lambda i: (i, 0)),
        ],
        out_shape=[
            jax.ShapeDtypeStruct((n, f), f32),
            jax.ShapeDtypeStruct((n, 1), f32),
        ],
    )(D, X)

    w1t = jnp.transpose(W1)
    w2t = jnp.transpose(W2)
    w3t = jnp.transpose(W3)
    b1r = jnp.reshape(b1, (1, h1))
    b2r = jnp.reshape(b2, (1, h2))
    b3r = jnp.reshape(b3, (1, c))

    hs, rsdj0 = pl.pallas_call(
        _layer1_kernel,
        grid=(n // BM, n // BK),
        in_specs=[
            pl.BlockSpec((BM, BK), lambda i, k: (i, k)),
            pl.BlockSpec((BK, f), lambda i, k: (k, 0)),
            pl.BlockSpec((BK, 1), lambda i, k: (k, 0)),
            pl.BlockSpec((BM, 1), lambda i, k: (i, 0)),
            pl.BlockSpec((f, h1), lambda i, k: (0, 0)),
            pl.BlockSpec((1, h1), lambda i, k: (0, 0)),
        ],
        out_specs=[
            pl.BlockSpec((BM, h1), lambda i, k: (i, 0)),
            pl.BlockSpec((BM, 1), lambda i, k: (i, 0)),
        ],
        out_shape=[
            jax.ShapeDtypeStruct((n, h1), f32),
            jax.ShapeDtypeStruct((n, 1), f32),
        ],
        scratch_shapes=[
            pltpu.VMEM((BM, f), f32),
            pltpu.VMEM((BM, 1), f32),
            pltpu.VMEM((BM, 1), jnp.int32),
        ],
    )(A, xs, rsdeg, rsdeg, w1t, b1r)

    out = pl.pallas_call(
        _layer2_kernel,
        grid=(n // BM, n // BK),
        in_specs=[
            pl.BlockSpec((BM, BK), lambda i, k: (i, k)),
            pl.BlockSpec((BK, h1), lambda i, k: (k, 0)),
            pl.BlockSpec((BM, 1), lambda i, k: (i, 0)),
            pl.BlockSpec((h1, h2), lambda i, k: (0, 0)),
            pl.BlockSpec((1, h2), lambda i, k: (0, 0)),
            pl.BlockSpec((h2, c), lambda i, k: (0, 0)),
            pl.BlockSpec((1, c), lambda i, k: (0, 0)),
        ],
        out_specs=pl.BlockSpec((BM, c), lambda i, k: (i, 0)),
        out_shape=jax.ShapeDtypeStruct((n, c), f32),
        scratch_shapes=[pltpu.VMEM((BM, h1), f32)],
    )(A, hs, rsdj0, w2t, b2r, w3t, b3r)

    return out
